# baseline (speedup 1.0000x reference)
import numpy as np

import jax
import jax.numpy as jnp
from jax import lax
from jax.experimental import pallas as pl
from jax.experimental.pallas import tpu as pltpu
from jax.experimental.pallas import tpu_sc as plsc

_B, _N, _D = 8, 1024, 256
_R2 = 4.0
_EPS = 1e-8
_NC, _NS = 2, 16
_NW = _NC * _NS
_KCAP = 256
_CH = 16
_PROW = 33


def _build_triu():
    j = np.arange(32)
    nbase = (j & 7) * 128 + (j >> 3) * 32
    r = np.arange(32)[:, None]
    c = np.arange(_N)[None, :]
    t = np.zeros((32, _N), np.uint32)
    for jj in range(32):
        t |= (c > (nbase[jj] + r)).astype(np.uint32) << np.uint32(jj)
    return t.view(np.int32)


_TRIU_NP = _build_triu()


def _norm_body(d_ref, out_ref):
    d = d_ref[0]
    nrm = jnp.maximum(jnp.sqrt(jnp.sum(d * d, axis=1, keepdims=True)), _EPS)
    out_ref[...] = d / nrm


def _mask_body(fac_ref, invis_ref, triu_ref, ps_ref, pdT_ref, packed_ref):
    p = pl.program_id(0)

    fx = fac_ref[0]
    fy = fac_ref[1]
    ps = ps_ref[0]
    psx = fx * (ps[:, 0:1] + 1.0)
    psy = fy * (ps[:, 1:2] + 1.0)
    pdT = pdT_ref[0, 0]
    pdx = fx * (pdT[0:1, :] + 1.0)
    pdy = fy * (pdT[1:2, :] + 1.0)
    ones_c = jnp.ones((_N, 1), jnp.float32)
    ones_r = jnp.ones((1, _N), jnp.float32)
    pmat = jnp.concatenate(
        [psx, psy, psx * psx + psy * psy, ones_c], axis=1)
    qmat = jnp.concatenate(
        [-2.0 * pdx, -2.0 * pdy, ones_r, pdx * pdx + pdy * pdy], axis=0)
    d2 = lax.dot_general(pmat, qmat, (((1,), (0,)), ((), ())),
                         precision=lax.Precision.HIGHEST,
                         preferred_element_type=jnp.float32)

    mi = (d2 <= _R2).astype(jnp.int32)

    w3 = mi[0:128, :] << 0
    for k in range(1, 8):
        w3 = w3 | (mi[k * 128:(k + 1) * 128, :] << k)
    w4 = w3[0:32, :]
    for q in range(1, 4):
        w4 = w4 | (w3[q * 32:(q + 1) * 32, :] << (8 * q))

    bs = invis_ref[0:1, :]
    bd = invis_ref[1:2, :]
    nn = invis_ref[2:3, :]
    pm = (bs * _B + bd) == p
    rk = nn & 31
    jk = ((nn >> 5) & 3) * 8 + (nn >> 7)
    riota = lax.broadcasted_iota(jnp.int32, (32, 1), 0)
    onehot = jnp.where(pm & (riota == rk),
                       jnp.left_shift(jnp.int32(1), jk),
                       jnp.zeros((32, 512), jnp.int32))
    v = onehot
    sz = 256
    while sz >= 1:
        v = v[:, 0:sz] | v[:, sz:2 * sz]
        sz //= 2
    visw = jnp.invert(v)

    w4 = w4 & triu_ref[...] & visw

    t = w4[:, 0:128]
    for j in range(1, 8):
        t = t | w4[:, j * 128:(j + 1) * 128]
    tnz = jnp.where(t != 0, 1, 0).astype(jnp.int32)
    s = tnz[0:1, :] << 0
    for r in range(1, 32):
        s = s | (tnz[r:r + 1, :] << r)

    packed_ref[0, 0:32, :] = w4
    packed_ref[0, 32:33, :] = jnp.concatenate(
        [s, jnp.zeros((1, _N - 128), jnp.int32)], axis=1)


def _ctz(w):
    low = w & (jnp.uint32(0) - w)
    f = low.astype(jnp.float32)
    bits = lax.bitcast_convert_type(f, jnp.int32)
    return (bits >> 23) - 127


def _popcount(w):
    w = w - ((w >> 1) & jnp.uint32(0x55555555))
    w = (w & jnp.uint32(0x33333333)) + ((w >> 2) & jnp.uint32(0x33333333))
    w = (w + (w >> 4)) & jnp.uint32(0x0F0F0F0F)
    return ((w * jnp.uint32(0x01010101)) >> 24).astype(jnp.int32)


def _sc_body(packed_hbm, nd_hbm, out_hbm, cnt_hbm,
             mbuf, sbuf, rows_s, rows_d, accv, cntv, isrc, idst, scnt,
             sem1, sem2):
    wid = lax.axis_index("s") * _NC + lax.axis_index("c")
    accv[...] = jnp.zeros((16,), jnp.float32)
    scnt[1] = 0

    def _word(ref, q):
        return ref[pl.ds(q, 16)][0]

    def do_pair(pp, _):
        p = wid * 2 + pp
        a = p >> 3
        b = p & 7
        pltpu.sync_copy(packed_hbm.at[p, pl.ds(0, 32 * _N)],
                        mbuf.at[pl.ds(0, 32 * _N)])
        pltpu.sync_copy(packed_hbm.at[p, pl.ds(32 * _N, _N)], sbuf)
        scnt[0] = 0

        def append(n, m):
            k = scnt[0]
            kk = jnp.minimum(k, _KCAP - 1)
            isrc[kk] = b * _N + n
            idst[kk] = a * _N + m
            scnt[0] = k + 1

        def col_body(c7, carry):
            sw = _word(sbuf, c7).astype(jnp.uint32)

            @pl.when(sw != jnp.uint32(0))
            def _cols():
                def rows_body(_i, w):
                    r = _ctz(w)
                    for j in range(8):
                        wj = _word(mbuf, r * _N + c7 + 128 * j)
                        wj = wj.astype(jnp.uint32)
                        m = c7 + 128 * j

                        @pl.when(wj != jnp.uint32(0))
                        def _bits(wj=wj, m=m):
                            def bits_body(_t, u):
                                j2 = _ctz(u)
                                n = ((j2 & 7) << 7) + ((j2 >> 3) << 5) + r
                                append(n, m)
                                return u & (u - jnp.uint32(1))

                            lax.fori_loop(0, _popcount(wj), bits_body, wj)
                    return w & (w - jnp.uint32(1))

                lax.fori_loop(0, _popcount(sw), rows_body, sw)

            return carry

        lax.fori_loop(0, 128, col_body, 0)

        cnt = jnp.minimum(scnt[0], _KCAP)
        scnt[1] = scnt[1] + scnt[0]
        lanes = lax.iota(jnp.int32, 16)
        for c in range(_KCAP // _CH):
            @pl.when(cnt > c * _CH)
            def _chunk(c=c):
                idx_s = jnp.zeros((16,), jnp.int32)
                idx_d = jnp.zeros((16,), jnp.int32)
                for j in range(_CH):
                    sel = lanes == j
                    idx_s = jnp.where(sel, jnp.full(
                        (16,), isrc[c * _CH + j] & (_B * _N - 1), jnp.int32),
                        idx_s)
                    idx_d = jnp.where(sel, jnp.full(
                        (16,), idst[c * _CH + j] & (_B * _N - 1), jnp.int32),
                        idx_d)
                d1 = pltpu.async_copy(nd_hbm.at[idx_s], rows_s, sem1)
                d2 = pltpu.async_copy(nd_hbm.at[idx_d], rows_d, sem2)
                d1.wait()
                d2.wait()
                nv = jnp.minimum(cnt - c * _CH, _CH)

                def dot_body(i, acc):
                    for k in range(_D // 16):
                        acc = acc + (rows_s[i, pl.ds(k * 16, 16)] *
                                     rows_d[i, pl.ds(k * 16, 16)])
                    return acc

                accv[...] = lax.fori_loop(0, nv, dot_body, accv[...])

        return 0

    lax.fori_loop(0, 2, do_pair, 0)
    pltpu.sync_copy(accv, out_hbm.at[wid])
    cntv[...] = jnp.full((16,), scnt[1], jnp.int32)
    pltpu.sync_copy(cntv, cnt_hbm.at[wid])


def _sc_call():
    return pl.kernel(
        _sc_body,
        out_type=[jax.ShapeDtypeStruct((_NW, 16), jnp.float32),
                  jax.ShapeDtypeStruct((_NW, 16), jnp.int32)],
        mesh=plsc.VectorSubcoreMesh(core_axis_name="c", subcore_axis_name="s",
                                    num_cores=_NC, num_subcores=_NS),
        scratch_types=[
            pltpu.VMEM((32 * _N + 16,), jnp.int32),
            pltpu.VMEM((_N,), jnp.int32),
            pltpu.VMEM((_CH, _D), jnp.float32),
            pltpu.VMEM((_CH, _D), jnp.float32),
            pltpu.VMEM((16,), jnp.float32),
            pltpu.VMEM((16,), jnp.int32),
            pltpu.SMEM((_KCAP,), jnp.int32),
            pltpu.SMEM((_KCAP,), jnp.int32),
            pltpu.SMEM((4,), jnp.int32),
            pltpu.SemaphoreType.DMA,
            pltpu.SemaphoreType.DMA,
        ],
    )


def kernel(descriptors, pts_src, pts_dst, invis_idx, height, width):
    fac = jnp.stack([(width - 1) * 0.5, (height - 1) * 0.5]).astype(jnp.float32)
    pdT = pts_dst.transpose(0, 1, 3, 2)
    invis = invis_idx.astype(jnp.int32)
    triu_words = jnp.asarray(_TRIU_NP)

    nd = pl.pallas_call(
        _norm_body,
        grid=(_B,),
        in_specs=[pl.BlockSpec((1, _N, _D), lambda b: (b, 0, 0))],
        out_specs=pl.BlockSpec((_N, _D), lambda b: (b, 0)),
        out_shape=jax.ShapeDtypeStruct((_B * _N, _D), jnp.float32),
    )(descriptors)

    packed = pl.pallas_call(
        _mask_body,
        grid=(_B * _B,),
        in_specs=[
            pl.BlockSpec(memory_space=pltpu.SMEM),
            pl.BlockSpec((3, 512), lambda p: (0, 0)),
            pl.BlockSpec((32, _N), lambda p: (0, 0)),
            pl.BlockSpec((1, _N, 2), lambda p: (p % _B, 0, 0)),
            pl.BlockSpec((1, 1, 2, _N), lambda p: (p // _B, p % _B, 0, 0)),
        ],
        out_specs=pl.BlockSpec((1, _PROW, _N), lambda p: (p, 0, 0)),
        out_shape=jax.ShapeDtypeStruct((_B * _B, _PROW, _N), jnp.int32),
    )(fac, invis, triu_words, pts_src, pdT)

    partial_cos, partial_cnt = _sc_call()(
        packed.reshape(_B * _B, _PROW * _N), nd)
    total = jnp.sum(partial_cnt[:, 0]).astype(jnp.float32)
    return (total - jnp.sum(partial_cos)) / total


# speedup vs baseline: 1.4511x; 1.4511x over previous
import numpy as np

import jax
import jax.numpy as jnp
from jax import lax
from jax.experimental import pallas as pl
from jax.experimental.pallas import tpu as pltpu
from jax.experimental.pallas import tpu_sc as plsc

_B, _N, _D = 8, 1024, 256
_R2 = 4.0
_EPS = 1e-8
_NC, _NS = 2, 16
_NW = _NC * _NS
_KCAP = 256
_CH = 16
_PROW = 33


def _build_triu():
    j = np.arange(32)
    nbase = (j & 7) * 128 + (j >> 3) * 32
    r = np.arange(32)[:, None]
    c = np.arange(_N)[None, :]
    t = np.zeros((32, _N), np.uint32)
    for jj in range(32):
        t |= (c > (nbase[jj] + r)).astype(np.uint32) << np.uint32(jj)
    return t.view(np.int32)


_TRIU_NP = _build_triu()


def _norm_body(d_ref, out_ref):
    d = d_ref[0]
    nrm = jnp.maximum(jnp.sqrt(jnp.sum(d * d, axis=1, keepdims=True)), _EPS)
    out_ref[...] = d / nrm


def _mask_body(fac_ref, invis_ref, triu_ref, ps_ref, pdT_ref, packed_ref):
    p = pl.program_id(0)

    fx = fac_ref[0]
    fy = fac_ref[1]
    ps = ps_ref[0]
    psx = fx * (ps[:, 0:1] + 1.0)
    psy = fy * (ps[:, 1:2] + 1.0)
    pdT = pdT_ref[0, 0]
    pdx = fx * (pdT[0:1, :] + 1.0)
    pdy = fy * (pdT[1:2, :] + 1.0)
    a2 = psx * psx + psy * psy
    b2 = pdx * pdx + pdy * pdy
    ab = psx * pdx + psy * pdy
    d2 = (a2 + b2) - 2.0 * ab

    mi = (d2 <= _R2).astype(jnp.int32)

    w3 = mi[0:128, :] << 0
    for k in range(1, 8):
        w3 = w3 | (mi[k * 128:(k + 1) * 128, :] << k)
    w4 = w3[0:32, :]
    for q in range(1, 4):
        w4 = w4 | (w3[q * 32:(q + 1) * 32, :] << (8 * q))

    bs = invis_ref[0:1, :]
    bd = invis_ref[1:2, :]
    nn = invis_ref[2:3, :]
    pm = (bs * _B + bd) == p
    rk = nn & 31
    jk = ((nn >> 5) & 3) * 8 + (nn >> 7)
    riota = lax.broadcasted_iota(jnp.int32, (32, 1), 0)
    onehot = jnp.where(pm & (riota == rk),
                       jnp.left_shift(jnp.int32(1), jk),
                       jnp.zeros((32, 512), jnp.int32))
    v = onehot
    sz = 256
    while sz >= 1:
        v = v[:, 0:sz] | v[:, sz:2 * sz]
        sz //= 2
    visw = jnp.invert(v)

    w4 = w4 & triu_ref[...] & visw

    t = w4[:, 0:128]
    for j in range(1, 8):
        t = t | w4[:, j * 128:(j + 1) * 128]
    tnz = jnp.where(t != 0, 1, 0).astype(jnp.int32)
    s = tnz[0:1, :] << 0
    for r in range(1, 32):
        s = s | (tnz[r:r + 1, :] << r)

    packed_ref[0, 0:32, :] = w4
    packed_ref[0, 32:33, :] = jnp.concatenate(
        [s, jnp.zeros((1, _N - 128), jnp.int32)], axis=1)


def _ctz(w):
    low = w & (jnp.uint32(0) - w)
    f = low.astype(jnp.float32)
    bits = lax.bitcast_convert_type(f, jnp.int32)
    return (bits >> 23) - 127


def _popcount(w):
    w = w - ((w >> 1) & jnp.uint32(0x55555555))
    w = (w & jnp.uint32(0x33333333)) + ((w >> 2) & jnp.uint32(0x33333333))
    w = (w + (w >> 4)) & jnp.uint32(0x0F0F0F0F)
    return ((w * jnp.uint32(0x01010101)) >> 24).astype(jnp.int32)


def _sc_body(packed_hbm, nd_hbm, out_hbm, cnt_hbm,
             mbuf, sbuf, rows_s, rows_d, accv, cntv, isrc, idst, scnt,
             sem1, sem2):
    wid = lax.axis_index("s") * _NC + lax.axis_index("c")
    accv[...] = jnp.zeros((16,), jnp.float32)
    scnt[1] = 0

    def _word(ref, q):
        return ref[pl.ds(q, 16)][0]

    def do_pair(pp, _):
        p = wid * 2 + pp
        a = p >> 3
        b = p & 7
        pltpu.sync_copy(packed_hbm.at[p, pl.ds(0, 32 * _N)],
                        mbuf.at[pl.ds(0, 32 * _N)])
        pltpu.sync_copy(packed_hbm.at[p, pl.ds(32 * _N, _N)], sbuf)
        scnt[0] = 0

        def append(n, m):
            k = scnt[0]
            kk = jnp.minimum(k, _KCAP - 1)
            isrc[kk] = b * _N + n
            idst[kk] = a * _N + m
            scnt[0] = k + 1

        def col_body(c7, carry):
            sw = _word(sbuf, c7).astype(jnp.uint32)

            @pl.when(sw != jnp.uint32(0))
            def _cols():
                def rows_body(_i, w):
                    r = _ctz(w)
                    for j in range(8):
                        wj = _word(mbuf, r * _N + c7 + 128 * j)
                        wj = wj.astype(jnp.uint32)
                        m = c7 + 128 * j

                        @pl.when(wj != jnp.uint32(0))
                        def _bits(wj=wj, m=m):
                            def bits_body(_t, u):
                                j2 = _ctz(u)
                                n = ((j2 & 7) << 7) + ((j2 >> 3) << 5) + r
                                append(n, m)
                                return u & (u - jnp.uint32(1))

                            lax.fori_loop(0, _popcount(wj), bits_body, wj)
                    return w & (w - jnp.uint32(1))

                lax.fori_loop(0, _popcount(sw), rows_body, sw)

            return carry

        lax.fori_loop(0, 128, col_body, 0)

        cnt = jnp.minimum(scnt[0], _KCAP)
        scnt[1] = scnt[1] + scnt[0]
        lanes = lax.iota(jnp.int32, 16)
        for c in range(_KCAP // _CH):
            @pl.when(cnt > c * _CH)
            def _chunk(c=c):
                idx_s = jnp.zeros((16,), jnp.int32)
                idx_d = jnp.zeros((16,), jnp.int32)
                for j in range(_CH):
                    sel = lanes == j
                    idx_s = jnp.where(sel, jnp.full(
                        (16,), isrc[c * _CH + j] & (_B * _N - 1), jnp.int32),
                        idx_s)
                    idx_d = jnp.where(sel, jnp.full(
                        (16,), idst[c * _CH + j] & (_B * _N - 1), jnp.int32),
                        idx_d)
                d1 = pltpu.async_copy(nd_hbm.at[idx_s], rows_s, sem1)
                d2 = pltpu.async_copy(nd_hbm.at[idx_d], rows_d, sem2)
                d1.wait()
                d2.wait()
                nv = jnp.minimum(cnt - c * _CH, _CH)

                def dot_body(i, acc):
                    for k in range(_D // 16):
                        acc = acc + (rows_s[i, pl.ds(k * 16, 16)] *
                                     rows_d[i, pl.ds(k * 16, 16)])
                    return acc

                accv[...] = lax.fori_loop(0, nv, dot_body, accv[...])

        return 0

    lax.fori_loop(0, 2, do_pair, 0)
    pltpu.sync_copy(accv, out_hbm.at[wid])
    cntv[...] = jnp.full((16,), scnt[1], jnp.int32)
    pltpu.sync_copy(cntv, cnt_hbm.at[wid])


def _sc_call():
    return pl.kernel(
        _sc_body,
        out_type=[jax.ShapeDtypeStruct((_NW, 16), jnp.float32),
                  jax.ShapeDtypeStruct((_NW, 16), jnp.int32)],
        mesh=plsc.VectorSubcoreMesh(core_axis_name="c", subcore_axis_name="s",
                                    num_cores=_NC, num_subcores=_NS),
        scratch_types=[
            pltpu.VMEM((32 * _N + 16,), jnp.int32),
            pltpu.VMEM((_N,), jnp.int32),
            pltpu.VMEM((_CH, _D), jnp.float32),
            pltpu.VMEM((_CH, _D), jnp.float32),
            pltpu.VMEM((16,), jnp.float32),
            pltpu.VMEM((16,), jnp.int32),
            pltpu.SMEM((_KCAP,), jnp.int32),
            pltpu.SMEM((_KCAP,), jnp.int32),
            pltpu.SMEM((4,), jnp.int32),
            pltpu.SemaphoreType.DMA,
            pltpu.SemaphoreType.DMA,
        ],
    )


def kernel(descriptors, pts_src, pts_dst, invis_idx, height, width):
    fac = jnp.stack([(width - 1) * 0.5, (height - 1) * 0.5]).astype(jnp.float32)
    pdT = pts_dst.transpose(0, 1, 3, 2)
    invis = invis_idx.astype(jnp.int32)
    triu_words = jnp.asarray(_TRIU_NP)

    nd = pl.pallas_call(
        _norm_body,
        grid=(_B,),
        in_specs=[pl.BlockSpec((1, _N, _D), lambda b: (b, 0, 0))],
        out_specs=pl.BlockSpec((_N, _D), lambda b: (b, 0)),
        out_shape=jax.ShapeDtypeStruct((_B * _N, _D), jnp.float32),
    )(descriptors)

    packed = pl.pallas_call(
        _mask_body,
        grid=(_B * _B,),
        in_specs=[
            pl.BlockSpec(memory_space=pltpu.SMEM),
            pl.BlockSpec((3, 512), lambda p: (0, 0)),
            pl.BlockSpec((32, _N), lambda p: (0, 0)),
            pl.BlockSpec((1, _N, 2), lambda p: (p % _B, 0, 0)),
            pl.BlockSpec((1, 1, 2, _N), lambda p: (p // _B, p % _B, 0, 0)),
        ],
        out_specs=pl.BlockSpec((1, _PROW, _N), lambda p: (p, 0, 0)),
        out_shape=jax.ShapeDtypeStruct((_B * _B, _PROW, _N), jnp.int32),
    )(fac, invis, triu_words, pts_src, pdT)

    partial_cos, partial_cnt = _sc_call()(
        packed.reshape(_B * _B, _PROW * _N), nd)
    total = jnp.sum(partial_cnt[:, 0]).astype(jnp.float32)
    return (total - jnp.sum(partial_cos)) / total


# speedup vs baseline: 1.7868x; 1.2313x over previous
import numpy as np

import jax
import jax.numpy as jnp
from jax import lax
from jax.experimental import pallas as pl
from jax.experimental.pallas import tpu as pltpu
from jax.experimental.pallas import tpu_sc as plsc

_B, _N, _D = 8, 1024, 256
_R2 = 4.0
_EPS = 1e-8
_NC, _NS = 2, 16
_NW = _NC * _NS
_KCAP = 256
_CH = 16
_PROW = 33


def _build_triu():
    j = np.arange(32)
    nbase = (j & 7) * 128 + (j >> 3) * 32
    r = np.arange(32)[:, None]
    c = np.arange(_N)[None, :]
    t = np.zeros((32, _N), np.uint32)
    for jj in range(32):
        t |= (c > (nbase[jj] + r)).astype(np.uint32) << np.uint32(jj)
    return t.view(np.int32)


_TRIU_NP = _build_triu()


def _norm_body(d_ref, out_ref):
    d = d_ref[0]
    nrm = jnp.maximum(jnp.sqrt(jnp.sum(d * d, axis=1, keepdims=True)), _EPS)
    out_ref[...] = d / nrm


def _mask_body(fac_ref, invis_ref, triu_ref, ps_ref, pdT_ref, packed_ref):
    p = pl.program_id(0)

    fx = fac_ref[0]
    fy = fac_ref[1]
    ps = ps_ref[0]
    psx = fx * (ps[:, 0:1] + 1.0)
    psy = fy * (ps[:, 1:2] + 1.0)
    pdT = pdT_ref[0, 0]
    pdx = fx * (pdT[0:1, :] + 1.0)
    pdy = fy * (pdT[1:2, :] + 1.0)
    a2 = psx * psx + psy * psy
    b2 = pdx * pdx + pdy * pdy

    bs = invis_ref[0:1, :]
    bd = invis_ref[1:2, :]
    nn = invis_ref[2:3, :]
    pm = (bs * _B + bd) == p
    rk = nn & 31
    jk = ((nn >> 5) & 3) * 8 + (nn >> 7)
    riota = lax.broadcasted_iota(jnp.int32, (32, 1), 0)
    onehot = jnp.where(pm & (riota == rk),
                       jnp.left_shift(jnp.int32(1), jk),
                       jnp.zeros((32, 512), jnp.int32))
    v = onehot
    sz = 256
    while sz >= 1:
        v = v[:, 0:sz] | v[:, sz:2 * sz]
        sz //= 2
    visw = jnp.invert(v)

    t = None
    for j in range(8):
        cs = slice(128 * j, 128 * (j + 1))
        pdx_j = pdx[:, cs]
        pdy_j = pdy[:, cs]
        b2_j = b2[:, cs]
        w3_j = None
        for k in range(j + 1):
            rs = slice(128 * k, 128 * (k + 1))
            ab = psx[rs] * pdx_j + psy[rs] * pdy_j
            d2 = (a2[rs] + b2_j) - 2.0 * ab
            bit = jnp.where(d2 <= _R2, jnp.int32(1 << k), jnp.int32(0))
            w3_j = bit if w3_j is None else (w3_j | bit)
        w4_j = w3_j[0:32, :]
        for q in range(1, 4):
            w4_j = w4_j | (w3_j[q * 32:(q + 1) * 32, :] << (8 * q))
        w4_j = w4_j & triu_ref[:, cs] & visw
        t = w4_j if t is None else (t | w4_j)
        packed_ref[0, 0:32, cs] = w4_j

    tnz = jnp.where(t != 0, 1, 0).astype(jnp.int32)
    s = tnz[0:1, :] << 0
    for r in range(1, 32):
        s = s | (tnz[r:r + 1, :] << r)

    packed_ref[0, 32:33, :] = jnp.concatenate(
        [s, jnp.zeros((1, _N - 128), jnp.int32)], axis=1)


def _ctz(w):
    low = w & (jnp.uint32(0) - w)
    f = low.astype(jnp.float32)
    bits = lax.bitcast_convert_type(f, jnp.int32)
    return (bits >> 23) - 127


def _popcount(w):
    w = w - ((w >> 1) & jnp.uint32(0x55555555))
    w = (w & jnp.uint32(0x33333333)) + ((w >> 2) & jnp.uint32(0x33333333))
    w = (w + (w >> 4)) & jnp.uint32(0x0F0F0F0F)
    return ((w * jnp.uint32(0x01010101)) >> 24).astype(jnp.int32)


def _sc_body(packed_hbm, nd_hbm, out_hbm, cnt_hbm,
             mbuf, sbuf, rows_s, rows_d, accv, cntv, isrc, idst, scnt,
             sem1, sem2):
    wid = lax.axis_index("s") * _NC + lax.axis_index("c")
    accv[...] = jnp.zeros((16,), jnp.float32)
    scnt[1] = 0

    def _word(ref, q):
        return ref[pl.ds(q, 16)][0]

    def do_pair(pp, _):
        p = wid * 2 + pp
        a = p >> 3
        b = p & 7
        pltpu.sync_copy(packed_hbm.at[p, pl.ds(0, 32 * _N)],
                        mbuf.at[pl.ds(0, 32 * _N)])
        pltpu.sync_copy(packed_hbm.at[p, pl.ds(32 * _N, _N)], sbuf)
        scnt[0] = 0

        def append(n, m):
            k = scnt[0]
            kk = jnp.minimum(k, _KCAP - 1)
            isrc[kk] = b * _N + n
            idst[kk] = a * _N + m
            scnt[0] = k + 1

        def col_body(c7, carry):
            sw = _word(sbuf, c7).astype(jnp.uint32)

            @pl.when(sw != jnp.uint32(0))
            def _cols():
                def rows_body(_i, w):
                    r = _ctz(w)
                    for j in range(8):
                        wj = _word(mbuf, r * _N + c7 + 128 * j)
                        wj = wj.astype(jnp.uint32)
                        m = c7 + 128 * j

                        @pl.when(wj != jnp.uint32(0))
                        def _bits(wj=wj, m=m):
                            def bits_body(_t, u):
                                j2 = _ctz(u)
                                n = ((j2 & 7) << 7) + ((j2 >> 3) << 5) + r
                                append(n, m)
                                return u & (u - jnp.uint32(1))

                            lax.fori_loop(0, _popcount(wj), bits_body, wj)
                    return w & (w - jnp.uint32(1))

                lax.fori_loop(0, _popcount(sw), rows_body, sw)

            return carry

        lax.fori_loop(0, 128, col_body, 0)

        cnt = jnp.minimum(scnt[0], _KCAP)
        scnt[1] = scnt[1] + scnt[0]
        lanes = lax.iota(jnp.int32, 16)
        for c in range(_KCAP // _CH):
            @pl.when(cnt > c * _CH)
            def _chunk(c=c):
                idx_s = jnp.zeros((16,), jnp.int32)
                idx_d = jnp.zeros((16,), jnp.int32)
                for j in range(_CH):
                    sel = lanes == j
                    idx_s = jnp.where(sel, jnp.full(
                        (16,), isrc[c * _CH + j] & (_B * _N - 1), jnp.int32),
                        idx_s)
                    idx_d = jnp.where(sel, jnp.full(
                        (16,), idst[c * _CH + j] & (_B * _N - 1), jnp.int32),
                        idx_d)
                d1 = pltpu.async_copy(nd_hbm.at[idx_s], rows_s, sem1)
                d2 = pltpu.async_copy(nd_hbm.at[idx_d], rows_d, sem2)
                d1.wait()
                d2.wait()
                nv = jnp.minimum(cnt - c * _CH, _CH)

                def dot_body(i, acc):
                    for k in range(_D // 16):
                        acc = acc + (rows_s[i, pl.ds(k * 16, 16)] *
                                     rows_d[i, pl.ds(k * 16, 16)])
                    return acc

                accv[...] = lax.fori_loop(0, nv, dot_body, accv[...])

        return 0

    lax.fori_loop(0, 2, do_pair, 0)
    pltpu.sync_copy(accv, out_hbm.at[wid])
    cntv[...] = jnp.full((16,), scnt[1], jnp.int32)
    pltpu.sync_copy(cntv, cnt_hbm.at[wid])


def _sc_call():
    return pl.kernel(
        _sc_body,
        out_type=[jax.ShapeDtypeStruct((_NW, 16), jnp.float32),
                  jax.ShapeDtypeStruct((_NW, 16), jnp.int32)],
        mesh=plsc.VectorSubcoreMesh(core_axis_name="c", subcore_axis_name="s",
                                    num_cores=_NC, num_subcores=_NS),
        scratch_types=[
            pltpu.VMEM((32 * _N + 16,), jnp.int32),
            pltpu.VMEM((_N,), jnp.int32),
            pltpu.VMEM((_CH, _D), jnp.float32),
            pltpu.VMEM((_CH, _D), jnp.float32),
            pltpu.VMEM((16,), jnp.float32),
            pltpu.VMEM((16,), jnp.int32),
            pltpu.SMEM((_KCAP,), jnp.int32),
            pltpu.SMEM((_KCAP,), jnp.int32),
            pltpu.SMEM((4,), jnp.int32),
            pltpu.SemaphoreType.DMA,
            pltpu.SemaphoreType.DMA,
        ],
    )


def kernel(descriptors, pts_src, pts_dst, invis_idx, height, width):
    fac = jnp.stack([(width - 1) * 0.5, (height - 1) * 0.5]).astype(jnp.float32)
    pdT = pts_dst.transpose(0, 1, 3, 2)
    invis = invis_idx.astype(jnp.int32)
    triu_words = jnp.asarray(_TRIU_NP)

    nd = pl.pallas_call(
        _norm_body,
        grid=(_B,),
        in_specs=[pl.BlockSpec((1, _N, _D), lambda b: (b, 0, 0))],
        out_specs=pl.BlockSpec((_N, _D), lambda b: (b, 0)),
        out_shape=jax.ShapeDtypeStruct((_B * _N, _D), jnp.float32),
    )(descriptors)

    packed = pl.pallas_call(
        _mask_body,
        grid=(_B * _B,),
        in_specs=[
            pl.BlockSpec(memory_space=pltpu.SMEM),
            pl.BlockSpec((3, 512), lambda p: (0, 0)),
            pl.BlockSpec((32, _N), lambda p: (0, 0)),
            pl.BlockSpec((1, _N, 2), lambda p: (p % _B, 0, 0)),
            pl.BlockSpec((1, 1, 2, _N), lambda p: (p // _B, p % _B, 0, 0)),
        ],
        out_specs=pl.BlockSpec((1, _PROW, _N), lambda p: (p, 0, 0)),
        out_shape=jax.ShapeDtypeStruct((_B * _B, _PROW, _N), jnp.int32),
    )(fac, invis, triu_words, pts_src, pdT)

    partial_cos, partial_cnt = _sc_call()(
        packed.reshape(_B * _B, _PROW * _N), nd)
    total = jnp.sum(partial_cnt[:, 0]).astype(jnp.float32)
    return (total - jnp.sum(partial_cos)) / total


# speedup vs baseline: 1.7931x; 1.0035x over previous
import numpy as np

import jax
import jax.numpy as jnp
from jax import lax
from jax.experimental import pallas as pl
from jax.experimental.pallas import tpu as pltpu
from jax.experimental.pallas import tpu_sc as plsc

_B, _N, _D = 8, 1024, 256
_R2 = 4.0
_EPS = 1e-8
_NC, _NS = 2, 16
_NW = _NC * _NS
_KCAP = 256
_CH = 16
_PROW = 33


def _build_triu():
    j = np.arange(32)
    nbase = (j & 7) * 128 + (j >> 3) * 32
    r = np.arange(32)[:, None]
    c = np.arange(_N)[None, :]
    t = np.zeros((32, _N), np.uint32)
    for jj in range(32):
        t |= (c > (nbase[jj] + r)).astype(np.uint32) << np.uint32(jj)
    return t.view(np.int32)


_TRIU_NP = _build_triu()


def _norm_body(d_ref, out_ref):
    d = d_ref[0]
    nrm = jnp.maximum(jnp.sqrt(jnp.sum(d * d, axis=1, keepdims=True)), _EPS)
    out_ref[...] = d / nrm


def _mask_body(fac_ref, invis_ref, triu_ref, ps_ref, pdT_ref, packed_ref):
    p = pl.program_id(0)

    fx = fac_ref[0]
    fy = fac_ref[1]
    ps = ps_ref[0]
    psx = fx * (ps[:, 0:1] + 1.0)
    psy = fy * (ps[:, 1:2] + 1.0)
    pdT = pdT_ref[0, 0]
    pdx = fx * (pdT[0:1, :] + 1.0)
    pdy = fy * (pdT[1:2, :] + 1.0)
    a2 = psx * psx + psy * psy
    b2 = pdx * pdx + pdy * pdy

    bs = invis_ref[0:1, :]
    bd = invis_ref[1:2, :]
    nn = invis_ref[2:3, :]
    pm = (bs * _B + bd) == p
    rk = nn & 31
    jk = ((nn >> 5) & 3) * 8 + (nn >> 7)
    riota = lax.broadcasted_iota(jnp.int32, (32, 1), 0)
    onehot = jnp.where(pm & (riota == rk),
                       jnp.left_shift(jnp.int32(1), jk),
                       jnp.zeros((32, 512), jnp.int32))
    v = onehot
    sz = 256
    while sz >= 1:
        v = v[:, 0:sz] | v[:, sz:2 * sz]
        sz //= 2
    visw = jnp.invert(v)

    t = None
    for j in range(8):
        cs = slice(128 * j, 128 * (j + 1))
        pdx_j = pdx[:, cs]
        pdy_j = pdy[:, cs]
        b2_j = b2[:, cs]
        w3_j = None
        for k in range(j + 1):
            rs = slice(128 * k, 128 * (k + 1))
            ab = psx[rs] * pdx_j + psy[rs] * pdy_j
            d2 = (a2[rs] + b2_j) - 2.0 * ab
            bit = jnp.where(d2 <= _R2, jnp.int32(1 << k), jnp.int32(0))
            w3_j = bit if w3_j is None else (w3_j | bit)
        w4_j = w3_j[0:32, :]
        for q in range(1, 4):
            w4_j = w4_j | (w3_j[q * 32:(q + 1) * 32, :] << (8 * q))
        w4_j = w4_j & triu_ref[:, cs] & visw
        t = w4_j if t is None else (t | w4_j)
        packed_ref[0, 0:32, cs] = w4_j

    tnz = jnp.where(t != 0, 1, 0).astype(jnp.int32)
    s = tnz[0:1, :] << 0
    for r in range(1, 32):
        s = s | (tnz[r:r + 1, :] << r)

    packed_ref[0, 32:33, :] = jnp.concatenate(
        [s, jnp.zeros((1, _N - 128), jnp.int32)], axis=1)


def _ctz(w):
    low = w & (jnp.uint32(0) - w)
    f = low.astype(jnp.float32)
    bits = lax.bitcast_convert_type(f, jnp.int32)
    return (bits >> 23) - 127


def _popcount(w):
    w = w - ((w >> 1) & jnp.uint32(0x55555555))
    w = (w & jnp.uint32(0x33333333)) + ((w >> 2) & jnp.uint32(0x33333333))
    w = (w + (w >> 4)) & jnp.uint32(0x0F0F0F0F)
    return ((w * jnp.uint32(0x01010101)) >> 24).astype(jnp.int32)


def _sc_body(packed_hbm, nd_hbm, out_hbm, cnt_hbm,
             mbuf, sbuf, rows_s, rows_d, rows_s2, rows_d2, accv, cntv,
             isrc, idst, scnt, sem1, sem2, sem3, sem4):
    wid = lax.axis_index("s") * _NC + lax.axis_index("c")
    accv[...] = jnp.zeros((16,), jnp.float32)
    scnt[1] = 0

    def _word(ref, q):
        return ref[pl.ds(q, 16)][0]

    def do_pair(pp, _):
        p = wid * 2 + pp
        a = p >> 3
        b = p & 7
        pltpu.sync_copy(packed_hbm.at[p, pl.ds(0, 32 * _N)],
                        mbuf.at[pl.ds(0, 32 * _N)])
        pltpu.sync_copy(packed_hbm.at[p, pl.ds(32 * _N, _N)], sbuf)
        scnt[0] = 0

        def append(n, m):
            k = scnt[0]
            kk = jnp.minimum(k, _KCAP - 1)
            isrc[kk] = b * _N + n
            idst[kk] = a * _N + m
            scnt[0] = k + 1

        def col_body(c7, carry):
            sw = _word(sbuf, c7).astype(jnp.uint32)

            @pl.when(sw != jnp.uint32(0))
            def _cols():
                def rows_body(_i, w):
                    r = _ctz(w)
                    for j in range(8):
                        wj = _word(mbuf, r * _N + c7 + 128 * j)
                        wj = wj.astype(jnp.uint32)
                        m = c7 + 128 * j

                        @pl.when(wj != jnp.uint32(0))
                        def _bits(wj=wj, m=m):
                            def bits_body(_t, u):
                                j2 = _ctz(u)
                                n = ((j2 & 7) << 7) + ((j2 >> 3) << 5) + r
                                append(n, m)
                                return u & (u - jnp.uint32(1))

                            lax.fori_loop(0, _popcount(wj), bits_body, wj)
                    return w & (w - jnp.uint32(1))

                lax.fori_loop(0, _popcount(sw), rows_body, sw)

            return carry

        lax.fori_loop(0, 128, col_body, 0)

        cnt = jnp.minimum(scnt[0], _KCAP)
        scnt[1] = scnt[1] + scnt[0]
        lanes = lax.iota(jnp.int32, 16)
        zidx = jnp.zeros((16,), jnp.int32)
        bufs = ((rows_s, rows_d, sem1, sem2), (rows_s2, rows_d2, sem3, sem4))
        nch = _KCAP // _CH

        def issue(c):
            rs, rd, ss, sd = bufs[c % 2]
            idx_s = jnp.zeros((16,), jnp.int32)
            idx_d = jnp.zeros((16,), jnp.int32)
            for j in range(_CH):
                sel = lanes == j
                idx_s = jnp.where(sel, jnp.full(
                    (16,), isrc[c * _CH + j] & (_B * _N - 1), jnp.int32),
                    idx_s)
                idx_d = jnp.where(sel, jnp.full(
                    (16,), idst[c * _CH + j] & (_B * _N - 1), jnp.int32),
                    idx_d)
            pltpu.async_copy(nd_hbm.at[idx_s], rs, ss)
            pltpu.async_copy(nd_hbm.at[idx_d], rd, sd)

        def drain_dot(c):
            rs, rd, ss, sd = bufs[c % 2]
            pltpu.make_async_copy(nd_hbm.at[zidx], rs, ss).wait()
            pltpu.make_async_copy(nd_hbm.at[zidx], rd, sd).wait()
            nv = jnp.minimum(cnt - c * _CH, _CH)

            def dot_body(i, acc):
                for k in range(_D // 16):
                    acc = acc + (rs[i, pl.ds(k * 16, 16)] *
                                 rd[i, pl.ds(k * 16, 16)])
                return acc

            accv[...] = lax.fori_loop(0, nv, dot_body, accv[...])

        for c in range(nch + 1):
            if c < nch:
                @pl.when(cnt > c * _CH)
                def _iss(c=c):
                    issue(c)
            if c >= 1:
                @pl.when(cnt > (c - 1) * _CH)
                def _drn(c=c):
                    drain_dot(c - 1)

        return 0

    lax.fori_loop(0, 2, do_pair, 0)
    pltpu.sync_copy(accv, out_hbm.at[wid])
    cntv[...] = jnp.full((16,), scnt[1], jnp.int32)
    pltpu.sync_copy(cntv, cnt_hbm.at[wid])


def _sc_call():
    return pl.kernel(
        _sc_body,
        out_type=[jax.ShapeDtypeStruct((_NW, 16), jnp.float32),
                  jax.ShapeDtypeStruct((_NW, 16), jnp.int32)],
        mesh=plsc.VectorSubcoreMesh(core_axis_name="c", subcore_axis_name="s",
                                    num_cores=_NC, num_subcores=_NS),
        scratch_types=[
            pltpu.VMEM((32 * _N + 16,), jnp.int32),
            pltpu.VMEM((_N,), jnp.int32),
            pltpu.VMEM((_CH, _D), jnp.float32),
            pltpu.VMEM((_CH, _D), jnp.float32),
            pltpu.VMEM((_CH, _D), jnp.float32),
            pltpu.VMEM((_CH, _D), jnp.float32),
            pltpu.VMEM((16,), jnp.float32),
            pltpu.VMEM((16,), jnp.int32),
            pltpu.SMEM((_KCAP,), jnp.int32),
            pltpu.SMEM((_KCAP,), jnp.int32),
            pltpu.SMEM((4,), jnp.int32),
            pltpu.SemaphoreType.DMA,
            pltpu.SemaphoreType.DMA,
            pltpu.SemaphoreType.DMA,
            pltpu.SemaphoreType.DMA,
        ],
    )


def kernel(descriptors, pts_src, pts_dst, invis_idx, height, width):
    fac = jnp.stack([(width - 1) * 0.5, (height - 1) * 0.5]).astype(jnp.float32)
    pdT = pts_dst.transpose(0, 1, 3, 2)
    invis = invis_idx.astype(jnp.int32)
    triu_words = jnp.asarray(_TRIU_NP)

    nd = pl.pallas_call(
        _norm_body,
        grid=(_B,),
        in_specs=[pl.BlockSpec((1, _N, _D), lambda b: (b, 0, 0))],
        out_specs=pl.BlockSpec((_N, _D), lambda b: (b, 0)),
        out_shape=jax.ShapeDtypeStruct((_B * _N, _D), jnp.float32),
    )(descriptors)

    packed = pl.pallas_call(
        _mask_body,
        grid=(_B * _B,),
        in_specs=[
            pl.BlockSpec(memory_space=pltpu.SMEM),
            pl.BlockSpec((3, 512), lambda p: (0, 0)),
            pl.BlockSpec((32, _N), lambda p: (0, 0)),
            pl.BlockSpec((1, _N, 2), lambda p: (p % _B, 0, 0)),
            pl.BlockSpec((1, 1, 2, _N), lambda p: (p // _B, p % _B, 0, 0)),
        ],
        out_specs=pl.BlockSpec((1, _PROW, _N), lambda p: (p, 0, 0)),
        out_shape=jax.ShapeDtypeStruct((_B * _B, _PROW, _N), jnp.int32),
    )(fac, invis, triu_words, pts_src, pdT)

    partial_cos, partial_cnt = _sc_call()(
        packed.reshape(_B * _B, _PROW * _N), nd)
    total = jnp.sum(partial_cnt[:, 0]).astype(jnp.float32)
    return (total - jnp.sum(partial_cos)) / total


# speedup vs baseline: 1.9503x; 1.0877x over previous
import functools

import numpy as np

import jax
import jax.numpy as jnp
from jax import lax
from jax.experimental import pallas as pl
from jax.experimental.pallas import tpu as pltpu
from jax.experimental.pallas import tpu_sc as plsc

_B, _N, _D = 8, 1024, 256
_R2 = 4.0
_EPS = 1e-8
_NC, _NS = 2, 16
_NW = _NC * _NS
_KCAP = 256
_CH = 16
_PROW = 33


def _build_triu():
    j = np.arange(32)
    nbase = (j & 7) * 128 + (j >> 3) * 32
    r = np.arange(32)[:, None]
    c = np.arange(_N)[None, :]
    t = np.zeros((32, _N), np.uint32)
    for jj in range(32):
        t |= (c > (nbase[jj] + r)).astype(np.uint32) << np.uint32(jj)
    return t.view(np.int32)


_TRIU_NP = _build_triu()


def _norm_body(d_ref, out_ref):
    d = d_ref[0]
    nrm = jnp.maximum(jnp.sqrt(jnp.sum(d * d, axis=1, keepdims=True)), _EPS)
    out_ref[...] = d / nrm


def _mask_body(base, fac_ref, invis_ref, triu_ref, ps_ref, pdT_ref,
               packed_ref):
    p = pl.program_id(0) + base

    fx = fac_ref[0]
    fy = fac_ref[1]
    ps = ps_ref[0]
    psx = fx * (ps[:, 0:1] + 1.0)
    psy = fy * (ps[:, 1:2] + 1.0)
    pdT = pdT_ref[0, 0]
    pdx = fx * (pdT[0:1, :] + 1.0)
    pdy = fy * (pdT[1:2, :] + 1.0)
    a2 = psx * psx + psy * psy
    b2 = pdx * pdx + pdy * pdy

    bs = invis_ref[0:1, :]
    bd = invis_ref[1:2, :]
    nn = invis_ref[2:3, :]
    pm = (bs * _B + bd) == p
    rk = nn & 31
    jk = ((nn >> 5) & 3) * 8 + (nn >> 7)
    riota = lax.broadcasted_iota(jnp.int32, (32, 1), 0)
    onehot = jnp.where(pm & (riota == rk),
                       jnp.left_shift(jnp.int32(1), jk),
                       jnp.zeros((32, 512), jnp.int32))
    v = onehot
    sz = 256
    while sz >= 1:
        v = v[:, 0:sz] | v[:, sz:2 * sz]
        sz //= 2
    visw = jnp.invert(v)

    t = None
    for j in range(8):
        cs = slice(128 * j, 128 * (j + 1))
        pdx_j = pdx[:, cs]
        pdy_j = pdy[:, cs]
        b2_j = b2[:, cs]
        w3_j = None
        for k in range(j + 1):
            rs = slice(128 * k, 128 * (k + 1))
            ab = psx[rs] * pdx_j + psy[rs] * pdy_j
            d2 = (a2[rs] + b2_j) - 2.0 * ab
            bit = jnp.where(d2 <= _R2, jnp.int32(1 << k), jnp.int32(0))
            w3_j = bit if w3_j is None else (w3_j | bit)
        w4_j = w3_j[0:32, :]
        for q in range(1, 4):
            w4_j = w4_j | (w3_j[q * 32:(q + 1) * 32, :] << (8 * q))
        w4_j = w4_j & triu_ref[:, cs] & visw
        t = w4_j if t is None else (t | w4_j)
        packed_ref[0, 0:32, cs] = w4_j

    tnz = jnp.where(t != 0, 1, 0).astype(jnp.int32)
    s = tnz[0:1, :] << 0
    for r in range(1, 32):
        s = s | (tnz[r:r + 1, :] << r)

    packed_ref[0, 32:33, :] = jnp.concatenate(
        [s, jnp.zeros((1, _N - 128), jnp.int32)], axis=1)


def _ctz(w):
    low = w & (jnp.uint32(0) - w)
    f = low.astype(jnp.float32)
    bits = lax.bitcast_convert_type(f, jnp.int32)
    return (bits >> 23) - 127


def _popcount(w):
    w = w - ((w >> 1) & jnp.uint32(0x55555555))
    w = (w & jnp.uint32(0x33333333)) + ((w >> 2) & jnp.uint32(0x33333333))
    w = (w + (w >> 4)) & jnp.uint32(0x0F0F0F0F)
    return ((w * jnp.uint32(0x01010101)) >> 24).astype(jnp.int32)


def _sc_body(base, packed_hbm, nd_hbm, out_hbm, cnt_hbm,
             mbuf, sbuf, rows_s, rows_d, rows_s2, rows_d2, accv, cntv,
             isrc, idst, scnt, sem1, sem2, sem3, sem4):
    wid = lax.axis_index("s") * _NC + lax.axis_index("c")
    accv[...] = jnp.zeros((16,), jnp.float32)

    def _word(ref, q):
        return ref[pl.ds(q, 16)][0]

    def do_pair(pp, _):
        pg = base + wid
        p = wid
        a = pg >> 3
        b = pg & 7
        pltpu.sync_copy(packed_hbm.at[p, pl.ds(0, 32 * _N)],
                        mbuf.at[pl.ds(0, 32 * _N)])
        pltpu.sync_copy(packed_hbm.at[p, pl.ds(32 * _N, _N)], sbuf)
        scnt[0] = 0

        def append(n, m):
            k = scnt[0]
            kk = jnp.minimum(k, _KCAP - 1)
            isrc[kk] = b * _N + n
            idst[kk] = a * _N + m
            scnt[0] = k + 1

        def col_body(c7, carry):
            sw = _word(sbuf, c7).astype(jnp.uint32)

            @pl.when(sw != jnp.uint32(0))
            def _cols():
                def rows_body(_i, w):
                    r = _ctz(w)
                    for j in range(8):
                        wj = _word(mbuf, r * _N + c7 + 128 * j)
                        wj = wj.astype(jnp.uint32)
                        m = c7 + 128 * j

                        @pl.when(wj != jnp.uint32(0))
                        def _bits(wj=wj, m=m):
                            def bits_body(_t, u):
                                j2 = _ctz(u)
                                n = ((j2 & 7) << 7) + ((j2 >> 3) << 5) + r
                                append(n, m)
                                return u & (u - jnp.uint32(1))

                            lax.fori_loop(0, _popcount(wj), bits_body, wj)
                    return w & (w - jnp.uint32(1))

                lax.fori_loop(0, _popcount(sw), rows_body, sw)

            return carry

        lax.fori_loop(0, 128, col_body, 0)

        cnt = jnp.minimum(scnt[0], _KCAP)
        lanes = lax.iota(jnp.int32, 16)
        zidx = jnp.zeros((16,), jnp.int32)
        bufs = ((rows_s, rows_d, sem1, sem2), (rows_s2, rows_d2, sem3, sem4))
        nch = _KCAP // _CH

        def issue(c):
            rs, rd, ss, sd = bufs[c % 2]
            idx_s = jnp.zeros((16,), jnp.int32)
            idx_d = jnp.zeros((16,), jnp.int32)
            for j in range(_CH):
                sel = lanes == j
                idx_s = jnp.where(sel, jnp.full(
                    (16,), isrc[c * _CH + j] & (_B * _N - 1), jnp.int32),
                    idx_s)
                idx_d = jnp.where(sel, jnp.full(
                    (16,), idst[c * _CH + j] & (_B * _N - 1), jnp.int32),
                    idx_d)
            pltpu.async_copy(nd_hbm.at[idx_s], rs, ss)
            pltpu.async_copy(nd_hbm.at[idx_d], rd, sd)

        def drain_dot(c):
            rs, rd, ss, sd = bufs[c % 2]
            pltpu.make_async_copy(nd_hbm.at[zidx], rs, ss).wait()
            pltpu.make_async_copy(nd_hbm.at[zidx], rd, sd).wait()
            nv = jnp.minimum(cnt - c * _CH, _CH)

            def dot_body(i, acc):
                for k in range(_D // 16):
                    acc = acc + (rs[i, pl.ds(k * 16, 16)] *
                                 rd[i, pl.ds(k * 16, 16)])
                return acc

            accv[...] = lax.fori_loop(0, nv, dot_body, accv[...])

        for c in range(nch + 1):
            if c < nch:
                @pl.when(cnt > c * _CH)
                def _iss(c=c):
                    issue(c)
            if c >= 1:
                @pl.when(cnt > (c - 1) * _CH)
                def _drn(c=c):
                    drain_dot(c - 1)

        return 0

    lax.fori_loop(0, 1, do_pair, 0)
    pltpu.sync_copy(accv, out_hbm.at[wid])
    cntv[...] = jnp.full((16,), scnt[0], jnp.int32)
    pltpu.sync_copy(cntv, cnt_hbm.at[wid])


def _sc_call(base):
    return pl.kernel(
        functools.partial(_sc_body, base),
        out_type=[jax.ShapeDtypeStruct((_NW, 16), jnp.float32),
                  jax.ShapeDtypeStruct((_NW, 16), jnp.int32)],
        mesh=plsc.VectorSubcoreMesh(core_axis_name="c", subcore_axis_name="s",
                                    num_cores=_NC, num_subcores=_NS),
        scratch_types=[
            pltpu.VMEM((32 * _N + 16,), jnp.int32),
            pltpu.VMEM((_N,), jnp.int32),
            pltpu.VMEM((_CH, _D), jnp.float32),
            pltpu.VMEM((_CH, _D), jnp.float32),
            pltpu.VMEM((_CH, _D), jnp.float32),
            pltpu.VMEM((_CH, _D), jnp.float32),
            pltpu.VMEM((16,), jnp.float32),
            pltpu.VMEM((16,), jnp.int32),
            pltpu.SMEM((_KCAP,), jnp.int32),
            pltpu.SMEM((_KCAP,), jnp.int32),
            pltpu.SMEM((4,), jnp.int32),
            pltpu.SemaphoreType.DMA,
            pltpu.SemaphoreType.DMA,
            pltpu.SemaphoreType.DMA,
            pltpu.SemaphoreType.DMA,
        ],
    )


def kernel(descriptors, pts_src, pts_dst, invis_idx, height, width):
    fac = jnp.stack([(width - 1) * 0.5, (height - 1) * 0.5]).astype(jnp.float32)
    pdT = pts_dst.transpose(0, 1, 3, 2)
    invis = invis_idx.astype(jnp.int32)
    triu_words = jnp.asarray(_TRIU_NP)

    nd = pl.pallas_call(
        _norm_body,
        grid=(_B,),
        in_specs=[pl.BlockSpec((1, _N, _D), lambda b: (b, 0, 0))],
        out_specs=pl.BlockSpec((_N, _D), lambda b: (b, 0)),
        out_shape=jax.ShapeDtypeStruct((_B * _N, _D), jnp.float32),
    )(descriptors)

    cos_parts, cnt_parts = [], []
    for base in (0, _NW):
        packed = pl.pallas_call(
            functools.partial(_mask_body, base),
            grid=(_NW,),
            in_specs=[
                pl.BlockSpec(memory_space=pltpu.SMEM),
                pl.BlockSpec((3, 512), lambda g: (0, 0)),
                pl.BlockSpec((32, _N), lambda g: (0, 0)),
                pl.BlockSpec((1, _N, 2),
                             lambda g, base=base: ((g + base) % _B, 0, 0)),
                pl.BlockSpec((1, 1, 2, _N),
                             lambda g, base=base: ((g + base) // _B,
                                                   (g + base) % _B, 0, 0)),
            ],
            out_specs=pl.BlockSpec((1, _PROW, _N), lambda g: (g, 0, 0)),
            out_shape=jax.ShapeDtypeStruct((_NW, _PROW, _N), jnp.int32),
        )(fac, invis, triu_words, pts_src, pdT)

        pcos, pcnt = _sc_call(base)(packed.reshape(_NW, _PROW * _N), nd)
        cos_parts.append(jnp.sum(pcos))
        cnt_parts.append(jnp.sum(pcnt[:, 0]))

    total = (cnt_parts[0] + cnt_parts[1]).astype(jnp.float32)
    return (total - (cos_parts[0] + cos_parts[1])) / total
